# baseline (speedup 1.0000x reference)
import jax
import jax.numpy as jnp
from jax.experimental import pallas as pl
from jax.experimental.pallas import tpu as pltpu

_LANES = 16384

_PI = 3.141592653589793
_HALF_PI = 1.5707963267948966
_INV_2PI = 0.15915494309189535
_PI2_HI = 6.2831854820251465
_PI2_LO = -1.7484556000744344e-07

_SIN_C = (
    -7.647163731819816e-13,
    1.60590438368216146e-10,
    -2.50521083854417187e-08,
    2.75573192239858906e-06,
    -1.98412698412698413e-04,
    8.33333333333333333e-03,
    -1.66666666666666667e-01,
    1.0,
)
_COS_C = (
    -1.13826115303901135e-11,
    2.08767569878680990e-09,
    -2.75573192239858882e-07,
    2.48015873015873016e-05,
    -1.38888888888888889e-03,
    4.16666666666666666e-02,
    -5.0e-01,
    1.0,
)
_ATAN_C = (
    0.0024682466246835066,
    -0.014458697067341986,
    0.03989956003883502,
    -0.07247950662107988,
    0.10507319786922154,
    -0.14164333375087812,
    0.199865374891388,
    -0.33332657852595815,
    0.9999999055457106,
)


def _poly(x, coeffs):
    acc = jnp.full_like(x, coeffs[0])
    for c in coeffs[1:]:
        acc = acc * x + c
    return acc


def _sincos(t):
    n = jnp.round(t * _INV_2PI)
    r = t - n * _PI2_HI
    r = r - n * _PI2_LO
    r2 = r * r
    return r * _poly(r2, _SIN_C), _poly(r2, _COS_C)


def _atan2(y, x):
    ax = jnp.abs(x)
    ay = jnp.abs(y)
    swap = ay > ax
    num = jnp.where(swap, ax, ay)
    den = jnp.where(swap, ay, ax)
    a = num / den
    z = a * _poly(a * a, _ATAN_C)
    z = jnp.where(swap, _HALF_PI - z, z)
    z = jnp.where(x < 0, _PI - z, z)
    return jnp.where(y < 0, -z, z)


def _dense(row):
    return row.reshape(8, row.shape[1] // 8)


def _row(tile):
    return tile.reshape(1, tile.shape[0] * tile.shape[1])


def _fwd(sc_ref, w1_ref, b2_ref, w2_ref, b3_ref, w3_ref, out_ref):
    st = sc_ref[...]
    sx, sy = st[0:1, :], st[1:2, :]
    ang = st[4:6, :]
    L = ang.shape[1]
    sin2d, cos2d = _sincos(ang.reshape(16, L // 8))
    sin2 = sin2d.reshape(2, L)
    cos2 = cos2d.reshape(2, L)
    sin_s, sin_c = sin2[0:1, :], sin2[1:2, :]
    cos_s, cos_c = cos2[0:1, :], cos2[1:2, :]
    h = jnp.concatenate([st[0:4, :], cos_s, sin_s, cos_c, sin_c, st[6:7, :]], axis=0)
    h1 = jnp.maximum(
        jnp.dot(w1_ref[...], h, preferred_element_type=jnp.float32), 0.0
    )
    h2 = jnp.maximum(
        jnp.dot(w2_ref[...], h1, preferred_element_type=jnp.float32) + b2_ref[...], 0.0
    )
    d = jnp.dot(w3_ref[...], h2, preferred_element_type=jnp.float32) + b3_ref[...]
    dc, ds = d[2:3, :], d[3:4, :]
    sin_sd, cos_sd = sin2d[0:8, :], cos2d[0:8, :]
    dcd, dsd = _dense(dc), _dense(ds)
    th = _row(_atan2(sin_sd * dcd + cos_sd * dsd, cos_sd * dcd - sin_sd * dsd))
    out_ref[...] = jnp.concatenate([sx + d[0:1, :], sy + d[1:2, :], th], axis=0)


def kernel(states, control_inputs, W1, b1, W2, b2, W3, b3):
    B, N, _ = states.shape
    BN = B * N
    L = min(_LANES, BN)
    s3 = states.reshape(BN, 3)
    c3 = control_inputs.reshape(BN, 3)
    sc = jnp.concatenate(
        [
            s3[:, 0:2],
            c3[:, 0:2],
            s3[:, 2:3],
            c3[:, 2:3],
            jnp.ones((BN, 1), jnp.float32),
            jnp.zeros((BN, 1), jnp.float32),
        ],
        axis=1,
    ).T
    w1aug = jnp.concatenate(
        [W1[jnp.array([0, 1, 4, 5, 2, 3, 6, 7]), :], b1[None, :]], axis=0
    ).T
    full = lambda r, c: pl.BlockSpec((r, c), lambda i: (0, 0))
    out = pl.pallas_call(
        _fwd,
        grid=(BN // L,),
        in_specs=[
            pl.BlockSpec((8, L), lambda i: (0, i)),
            full(64, 9),
            full(64, 1),
            full(64, 64),
            full(4, 1),
            full(4, 64),
        ],
        out_specs=pl.BlockSpec((3, L), lambda i: (0, i)),
        out_shape=jax.ShapeDtypeStruct((3, BN), jnp.float32),
        compiler_params=pltpu.CompilerParams(dimension_semantics=("parallel",)),
    )(
        sc,
        w1aug,
        b2.reshape(64, 1),
        W2.T,
        b3.reshape(4, 1),
        W3.T,
    )
    return out.T.reshape(B, N, 3)


# speedup vs baseline: 1.5043x; 1.5043x over previous
import jax
import jax.numpy as jnp
from jax.experimental import pallas as pl
from jax.experimental.pallas import tpu as pltpu

_LANES = 16384

_PI = 3.141592653589793
_HALF_PI = 1.5707963267948966
_INV_2PI = 0.15915494309189535
_PI2_HI = 6.2831854820251465
_PI2_LO = -1.7484556000744344e-07

_SIN_C = (
    -7.647163731819816e-13,
    1.60590438368216146e-10,
    -2.50521083854417187e-08,
    2.75573192239858906e-06,
    -1.98412698412698413e-04,
    8.33333333333333333e-03,
    -1.66666666666666667e-01,
    1.0,
)
_COS_C = (
    -1.13826115303901135e-11,
    2.08767569878680990e-09,
    -2.75573192239858882e-07,
    2.48015873015873016e-05,
    -1.38888888888888889e-03,
    4.16666666666666666e-02,
    -5.0e-01,
    1.0,
)
_ATAN_C = (
    0.0024682466246835066,
    -0.014458697067341986,
    0.03989956003883502,
    -0.07247950662107988,
    0.10507319786922154,
    -0.14164333375087812,
    0.199865374891388,
    -0.33332657852595815,
    0.9999999055457106,
)


def _poly(x, coeffs):
    acc = jnp.full_like(x, coeffs[0])
    for c in coeffs[1:]:
        acc = acc * x + c
    return acc


def _sincos(t):
    n = jnp.round(t * _INV_2PI)
    r = t - n * _PI2_HI
    r = r - n * _PI2_LO
    r2 = r * r
    return r * _poly(r2, _SIN_C), _poly(r2, _COS_C)


def _atan2(y, x):
    ax = jnp.abs(x)
    ay = jnp.abs(y)
    swap = ay > ax
    num = jnp.where(swap, ax, ay)
    den = jnp.where(swap, ay, ax)
    a = num / den
    z = a * _poly(a * a, _ATAN_C)
    z = jnp.where(swap, _HALF_PI - z, z)
    z = jnp.where(x < 0, _PI - z, z)
    return jnp.where(y < 0, -z, z)


def _dense(row):
    return row.reshape(8, row.shape[1] // 8)


def _row(tile):
    return tile.reshape(1, tile.shape[0] * tile.shape[1])


def _fwd(sc_ref, w1_ref, b2_ref, w2_ref, b3_ref, w3_ref, out_ref):
    st = sc_ref[...]
    sx, sy = st[0:1, :], st[1:2, :]
    ang = jnp.concatenate([st[2:3, :], st[5:6, :]], axis=0)
    L = ang.shape[1]
    sin2d, cos2d = _sincos(ang.reshape(16, L // 8))
    sin2 = sin2d.reshape(2, L)
    cos2 = cos2d.reshape(2, L)
    sin_s, sin_c = sin2[0:1, :], sin2[1:2, :]
    cos_s, cos_c = cos2[0:1, :], cos2[1:2, :]
    h = jnp.concatenate(
        [st[0:2, :], st[3:5, :], cos_s, sin_s, cos_c, sin_c, st[6:7, :]], axis=0
    )
    h1 = jnp.maximum(
        jnp.dot(w1_ref[...], h, preferred_element_type=jnp.float32), 0.0
    )
    h2 = jnp.maximum(
        jnp.dot(w2_ref[...], h1, preferred_element_type=jnp.float32) + b2_ref[...], 0.0
    )
    d = jnp.dot(w3_ref[...], h2, preferred_element_type=jnp.float32) + b3_ref[...]
    dc, ds = d[2:3, :], d[3:4, :]
    sin_sd, cos_sd = sin2d[0:8, :], cos2d[0:8, :]
    dcd, dsd = _dense(dc), _dense(ds)
    th = _row(_atan2(sin_sd * dcd + cos_sd * dsd, cos_sd * dcd - sin_sd * dsd))
    out_ref[...] = jnp.concatenate([sx + d[0:1, :], sy + d[1:2, :], th], axis=0)


def kernel(states, control_inputs, W1, b1, W2, b2, W3, b3):
    B, N, _ = states.shape
    BN = B * N
    L = min(_LANES, BN)
    sc = jnp.concatenate(
        [
            states.reshape(BN, 3),
            control_inputs.reshape(BN, 3),
            jnp.ones((BN, 1), jnp.float32),
            jnp.zeros((BN, 1), jnp.float32),
        ],
        axis=1,
    ).T
    w1aug = jnp.concatenate(
        [W1[jnp.array([0, 1, 4, 5, 2, 3, 6, 7]), :], b1[None, :]], axis=0
    ).T
    full = lambda r, c: pl.BlockSpec((r, c), lambda i: (0, 0))
    out = pl.pallas_call(
        _fwd,
        grid=(BN // L,),
        in_specs=[
            pl.BlockSpec((8, L), lambda i: (0, i)),
            full(64, 9),
            full(64, 1),
            full(64, 64),
            full(4, 1),
            full(4, 64),
        ],
        out_specs=pl.BlockSpec((3, L), lambda i: (0, i)),
        out_shape=jax.ShapeDtypeStruct((3, BN), jnp.float32),
        compiler_params=pltpu.CompilerParams(dimension_semantics=("parallel",)),
    )(
        sc,
        w1aug,
        b2.reshape(64, 1),
        W2.T,
        b3.reshape(4, 1),
        W3.T,
    )
    return out.T.reshape(B, N, 3)
